# baseline (speedup 1.0000x reference)
import functools

import jax
import jax.numpy as jnp
from jax import lax
from jax.experimental import pallas as pl
from jax.experimental.pallas import tpu as pltpu
from jax.experimental.pallas import tpu_sc as plsc

N = 10000
E = 320000
D = 128
KD = 27
NUM_GRAPHS = 8
N_CLASSES = 10
NCORE = 2
NSUB = 16
NWORK = NCORE * NSUB
EPT = E // NWORK
CH = 16
NCHUNK = EPT // CH
DCH = 80
DNCHUNK = EPT // DCH
NP = 10240
RPT = NP // NSUB
ZR = 32


def _xk_body(x_ref, w_ref, out_ref):
    out_ref[0] = jnp.dot(x_ref[...], w_ref[0],
                         preferred_element_type=jnp.float32)


def _xk(x, W):
    return pl.pallas_call(
        _xk_body,
        grid=(KD,),
        in_specs=[
            pl.BlockSpec((N, D), lambda k: (0, 0)),
            pl.BlockSpec((1, D, D), lambda k: (k, 0, 0)),
        ],
        out_specs=pl.BlockSpec((1, N, D), lambda k: (k, 0, 0)),
        out_shape=jax.ShapeDtypeStruct((KD, N, D), jnp.float32),
    )(x, W)


_EB = 6400


def _eprep_body(ps_ref, b_ref, g_ref):
    ps = ps_ref[...]
    p = ps * jnp.float32(2.0)
    lo = jnp.clip(jnp.floor(p), 0.0, 1.0)
    fr = p - lo
    srcf = ps_ref[3:4, :]
    for s in range(8):
        bx, by, bz = (s >> 0) & 1, (s >> 1) & 1, (s >> 2) & 1
        w = jnp.ones_like(srcf)
        wi = jnp.zeros_like(srcf)
        for d, bit in enumerate((bx, by, bz)):
            frd = fr[d:d + 1, :]
            lod = lo[d:d + 1, :]
            w = w * (frd if bit else (1.0 - frd))
            wi = wi + (lod + jnp.float32(bit)) * jnp.float32(3 ** d)
        gidx = wi * jnp.float32(N) + srcf
        b_ref[s:s + 1, :] = w
        g_ref[s:s + 1, :] = gidx.astype(jnp.int32)


def _eprep(ps8):
    return pl.pallas_call(
        _eprep_body,
        grid=(E // _EB,),
        in_specs=[pl.BlockSpec((8, _EB), lambda j: (0, j))],
        out_specs=[pl.BlockSpec((8, _EB), lambda j: (0, j)),
                   pl.BlockSpec((8, _EB), lambda j: (0, j))],
        out_shape=[jax.ShapeDtypeStruct((8, E), jnp.float32),
                   jax.ShapeDtypeStruct((8, E), jnp.int32)],
    )(ps8)


def _sc_body(xk_hbm, g8_hbm, b8_hbm, dst_hbm, out_hbm,
             gv, bv, dv, rows, msg, bounce, acc, sem):
    c = lax.axis_index("c")
    s = lax.axis_index("s")
    wid = s * NCORE + c

    zero16 = jnp.zeros((16,), jnp.float32)

    def zrow(r, _):
        for j in range(D // 16):
            bounce[r, pl.ds(j * 16, 16)] = zero16
        return 0

    lax.fori_loop(0, ZR, zrow, 0)

    def zcopy(k, _):
        pltpu.sync_copy(bounce, acc.at[pl.ds(s * RPT + k * ZR, ZR)])
        return 0

    lax.fori_loop(0, RPT // ZR, zcopy, 0)
    plsc.subcore_barrier()

    def chunk(i, _):
        base = wid * EPT + i * CH
        fbase = (wid * NCHUNK + i) * 8 * CH
        pltpu.sync_copy(g8_hbm.at[pl.ds(fbase, 8 * CH)], gv)
        pltpu.sync_copy(b8_hbm.at[pl.ds(fbase, 8 * CH)], bv)
        pltpu.sync_copy(dst_hbm.at[pl.ds(base, CH)], dv)
        pltpu.async_copy(xk_hbm.at[gv], rows, sem).wait()

        b16 = [bv[pl.ds(k * CH, CH)] for k in range(8)]

        def edge(j, _):
            jdx = jnp.zeros((16,), jnp.int32) + j
            regs = [zero16] * (D // 16)
            for k in range(8):
                bb = lax.gather(
                    b16[k], jdx[:, None],
                    lax.GatherDimensionNumbers(
                        offset_dims=(), collapsed_slice_dims=(0,),
                        start_index_map=(0,)),
                    (1,),
                    mode=lax.GatherScatterMode.PROMISE_IN_BOUNDS)
                for j2 in range(D // 16):
                    regs[j2] = regs[j2] + bb * rows[k * CH + j,
                                                    pl.ds(j2 * 16, 16)]
            for j2 in range(D // 16):
                msg[j, pl.ds(j2 * 16, 16)] = regs[j2]
            return 0

        lax.fori_loop(0, CH, edge, 0)
        pltpu.sync_copy(msg, acc.at[dv], add=True)
        return 0

    lax.fori_loop(0, NCHUNK, chunk, 0)
    plsc.subcore_barrier()

    def ocopy(k, _):
        r0 = s * RPT + k * ZR
        pltpu.sync_copy(acc.at[pl.ds(r0, ZR)], bounce)
        pltpu.sync_copy(bounce, out_hbm.at[c].at[pl.ds(r0, ZR)])
        return 0

    lax.fori_loop(0, RPT // ZR, ocopy, 0)


def _sc_mp(xk_flat, g8, b8, dst):
    mesh = plsc.VectorSubcoreMesh(core_axis_name="c", subcore_axis_name="s")
    f = pl.kernel(
        _sc_body,
        out_type=jax.ShapeDtypeStruct((NCORE, NP, D), jnp.float32),
        mesh=mesh,
        scratch_types=[
            pltpu.VMEM((8 * CH,), jnp.int32),
            pltpu.VMEM((8 * CH,), jnp.float32),
            pltpu.VMEM((CH,), jnp.int32),
            pltpu.VMEM((8 * CH, D), jnp.float32),
            pltpu.VMEM((CH, D), jnp.float32),
            pltpu.VMEM((ZR, D), jnp.float32),
            pltpu.VMEM_SHARED((NP, D), jnp.float32),
            pltpu.SemaphoreType.DMA,
        ],
    )
    return f(xk_flat, g8, b8, dst)


def _deg_body(dst_hbm, out_hbm, dv, ones_b, bounce, acc, sem):
    c = lax.axis_index("c")
    s = lax.axis_index("s")
    wid = s * NCORE + c
    zero16 = jnp.zeros((16,), jnp.float32)
    one16 = jnp.ones((16,), jnp.float32)

    def zrow(r, _):
        for j in range(D // 16):
            bounce[r, pl.ds(j * 16, 16)] = zero16
        return 0

    lax.fori_loop(0, ZR, zrow, 0)

    def orow(r, _):
        for j in range(D // 16):
            ones_b[r, pl.ds(j * 16, 16)] = one16
        return 0

    lax.fori_loop(0, DCH, orow, 0)

    def zcopy(k, _):
        pltpu.sync_copy(bounce, acc.at[pl.ds(s * RPT + k * ZR, ZR)])
        return 0

    lax.fori_loop(0, RPT // ZR, zcopy, 0)
    plsc.subcore_barrier()

    def chunk(i, _):
        base = wid * EPT + i * DCH
        pltpu.sync_copy(dst_hbm.at[pl.ds(base, DCH)], dv)
        pltpu.sync_copy(ones_b, acc.at[dv], add=True)
        return 0

    lax.fori_loop(0, DNCHUNK, chunk, 0)
    plsc.subcore_barrier()

    def ocopy(k, _):
        r0 = s * RPT + k * ZR
        pltpu.sync_copy(acc.at[pl.ds(r0, ZR)], bounce)
        pltpu.sync_copy(bounce, out_hbm.at[c].at[pl.ds(r0, ZR)])
        return 0

    lax.fori_loop(0, RPT // ZR, ocopy, 0)


def _sc_deg(dst):
    mesh = plsc.VectorSubcoreMesh(core_axis_name="c", subcore_axis_name="s")
    f = pl.kernel(
        _deg_body,
        out_type=jax.ShapeDtypeStruct((NCORE, NP, D), jnp.float32),
        mesh=mesh,
        scratch_types=[
            pltpu.VMEM((DCH,), jnp.int32),
            pltpu.VMEM((DCH, D), jnp.float32),
            pltpu.VMEM((ZR, D), jnp.float32),
            pltpu.VMEM_SHARED((NP, D), jnp.float32),
            pltpu.SemaphoreType.DMA,
        ],
    )
    return f(dst)


def _combine_body(acc_ref, dacc_ref, x_ref, root_ref, bias_ref, g_ref,
                  be_ref, out_ref):
    a = acc_ref[0][:N, :] + acc_ref[1][:N, :]
    deg = jnp.maximum(dacc_ref[0][:N, :1] + dacc_ref[1][:N, :1], 1.0)
    agg = a / deg
    z = agg + jnp.dot(x_ref[...], root_ref[...],
                      preferred_element_type=jnp.float32) + bias_ref[...]
    z = jnp.maximum(z, 0.0)
    m = jnp.mean(z, axis=0, keepdims=True)
    v = jnp.mean((z - m) ** 2, axis=0, keepdims=True)
    out_ref[...] = (z - m) * lax.rsqrt(v + 1e-5) * g_ref[...] + be_ref[...]


def _combine(acc, dacc, x, root, bias, g, be):
    return pl.pallas_call(
        _combine_body,
        out_shape=jax.ShapeDtypeStruct((N, D), jnp.float32),
    )(acc, dacc, x, root, bias[None, :], g[None, :], be[None, :])


def _head_body(h_ref, batch_ref, L1w_ref, L1b_ref, g3_ref, be3_ref,
               F1w_ref, F1b_ref, g4_ref, be4_ref, F2w_ref, F2b_ref,
               g5_ref, be5_ref, F3w_ref, F3b_ref, out_ref):
    h = h_ref[...]
    z = jnp.maximum(jnp.dot(h, L1w_ref[...],
                            preferred_element_type=jnp.float32)
                    + L1b_ref[...], 0.0)
    m = jnp.mean(z, axis=0, keepdims=True)
    v = jnp.mean((z - m) ** 2, axis=0, keepdims=True)
    z = (z - m) * lax.rsqrt(v + 1e-5) * g3_ref[...] + be3_ref[...]
    batch = batch_ref[...]
    neg = jnp.float32(-3.0e38)
    rows = []
    for g in range(NUM_GRAPHS):
        mask = (batch == g)
        rows.append(jnp.max(jnp.where(mask, z, neg), axis=0)[None, :])
    pooled = jnp.concatenate(rows, axis=0)

    def bn_small(o, gr, ber):
        mm = jnp.mean(o, axis=0, keepdims=True)
        vv = jnp.mean((o - mm) ** 2, axis=0, keepdims=True)
        return (o - mm) * lax.rsqrt(vv + 1e-5) * gr[...] + ber[...]

    o = jnp.maximum(jnp.dot(pooled, F1w_ref[...],
                            preferred_element_type=jnp.float32)
                    + F1b_ref[...], 0.0)
    o = bn_small(o, g4_ref, be4_ref)
    o = jnp.maximum(jnp.dot(o, F2w_ref[...],
                            preferred_element_type=jnp.float32)
                    + F2b_ref[...], 0.0)
    o = bn_small(o, g5_ref, be5_ref)
    o = jnp.dot(o, F3w_ref[...], preferred_element_type=jnp.float32) \
        + F3b_ref[...]
    omax = jnp.max(o, axis=1, keepdims=True)
    lse = jnp.log(jnp.sum(jnp.exp(o - omax), axis=1, keepdims=True)) + omax
    out_ref[...] = o - lse


def _head(h, batch, L1w, L1b, g3, be3, F1w, F1b, g4, be4, F2w, F2b,
          g5, be5, F3w, F3b):
    return pl.pallas_call(
        _head_body,
        out_shape=jax.ShapeDtypeStruct((NUM_GRAPHS, N_CLASSES), jnp.float32),
    )(h, batch[:, None], L1w, L1b[None, :], g3[None, :], be3[None, :],
      F1w, F1b[None, :], g4[None, :], be4[None, :], F2w, F2b[None, :],
      g5[None, :], be5[None, :], F3w, F3b[None, :])


def kernel(x, edge_index, pseudo, batch, W1, root1, b1, g1, be1, W2, root2,
           b2, g2, be2, L1w, L1b, g3, be3, F1w, F1b, g4, be4, F2w, F2b,
           g5, be5, F3w, F3b):
    src = edge_index[0]
    dst = edge_index[1]
    ps8 = jnp.concatenate(
        [pseudo.T, src.astype(jnp.float32)[None, :],
         jnp.zeros((4, E), jnp.float32)], axis=0)
    b8, g8 = _eprep(ps8)
    b8f = b8.reshape(8, NWORK, NCHUNK, CH).transpose(1, 2, 0, 3).reshape(-1)
    g8f = g8.reshape(8, NWORK, NCHUNK, CH).transpose(1, 2, 0, 3).reshape(-1)

    dacc = _sc_deg(dst)

    def layer(xin, W, root, bias, g, be):
        xk = _xk(xin, W).reshape(KD * N, D)
        acc = _sc_mp(xk, g8f, b8f, dst)
        return _combine(acc, dacc, xin, root, bias, g, be)

    x1 = layer(x, W1, root1, b1, g1, be1)
    x2 = layer(x1, W2, root2, b2, g2, be2)
    h = jnp.concatenate([x1, x2], axis=1)
    return _head(h, batch, L1w, L1b, g3, be3, F1w, F1b, g4, be4,
                 F2w, F2b, g5, be5, F3w, F3b)


# speedup vs baseline: 2.1102x; 2.1102x over previous
import functools

import jax
import jax.numpy as jnp
from jax import lax
from jax.experimental import pallas as pl
from jax.experimental.pallas import tpu as pltpu
from jax.experimental.pallas import tpu_sc as plsc

N = 10000
E = 320000
D = 128
KD = 27
NUM_GRAPHS = 8
N_CLASSES = 10
NCORE = 2
NSUB = 16
NWORK = NCORE * NSUB
EPT = E // NWORK
CH = 16
NCHUNK = EPT // CH
DCH = 80
DNCHUNK = EPT // DCH
NP = 10240
RPT = NP // NSUB
ZR = 32


def _xk_body(x_ref, w_ref, out_ref):
    out_ref[0] = jnp.dot(x_ref[...], w_ref[0],
                         preferred_element_type=jnp.float32)


def _xk(x, W):
    return pl.pallas_call(
        _xk_body,
        grid=(KD,),
        in_specs=[
            pl.BlockSpec((N, D), lambda k: (0, 0)),
            pl.BlockSpec((1, D, D), lambda k: (k, 0, 0)),
        ],
        out_specs=pl.BlockSpec((1, N, D), lambda k: (k, 0, 0)),
        out_shape=jax.ShapeDtypeStruct((KD, N, D), jnp.float32),
    )(x, W)


_EB = 6400


def _eprep_body(ps_ref, b_ref, g_ref):
    ps = ps_ref[...]
    p = ps * jnp.float32(2.0)
    lo = jnp.clip(jnp.floor(p), 0.0, 1.0)
    fr = p - lo
    srcf = ps_ref[3:4, :]
    for s in range(8):
        bx, by, bz = (s >> 0) & 1, (s >> 1) & 1, (s >> 2) & 1
        w = jnp.ones_like(srcf)
        wi = jnp.zeros_like(srcf)
        for d, bit in enumerate((bx, by, bz)):
            frd = fr[d:d + 1, :]
            lod = lo[d:d + 1, :]
            w = w * (frd if bit else (1.0 - frd))
            wi = wi + (lod + jnp.float32(bit)) * jnp.float32(3 ** d)
        gidx = wi * jnp.float32(N) + srcf
        b_ref[s:s + 1, :] = w
        g_ref[s:s + 1, :] = gidx.astype(jnp.int32)


def _eprep(ps8):
    return pl.pallas_call(
        _eprep_body,
        grid=(E // _EB,),
        in_specs=[pl.BlockSpec((8, _EB), lambda j: (0, j))],
        out_specs=[pl.BlockSpec((8, _EB), lambda j: (0, j)),
                   pl.BlockSpec((8, _EB), lambda j: (0, j))],
        out_shape=[jax.ShapeDtypeStruct((8, E), jnp.float32),
                   jax.ShapeDtypeStruct((8, E), jnp.int32)],
    )(ps8)


def _sc_body(xk_hbm, g8_hbm, b8_hbm, dst_hbm, out_hbm,
             gv, bv, dv, dsc, rows, msg, bounce, acc, isem, rsem, ssem):
    c = lax.axis_index("c")
    s = lax.axis_index("s")
    wid = s * NCORE + c
    zero16 = jnp.zeros((16,), jnp.float32)

    def zrow(r, _):
        for j in range(D // 16):
            bounce[r, pl.ds(j * 16, 16)] = zero16
        return 0

    lax.fori_loop(0, ZR, zrow, 0)

    def zcopy(k, _):
        pltpu.sync_copy(bounce, acc.at[pl.ds(s * RPT + k * ZR, ZR)])
        return 0

    lax.fori_loop(0, RPT // ZR, zcopy, 0)
    plsc.subcore_barrier()

    def fetch_idx(ci, sl):
        base = wid * EPT + ci * CH
        fbase = (wid * NCHUNK + ci) * 8 * CH
        pltpu.async_copy(g8_hbm.at[pl.ds(fbase, 8 * CH)], gv.at[sl], isem)
        pltpu.async_copy(b8_hbm.at[pl.ds(fbase, 8 * CH)], bv.at[sl], isem)
        pltpu.async_copy(dst_hbm.at[pl.ds(base, CH)], dv.at[sl], isem)

    def wait_idx(sl):
        pltpu.make_async_copy(g8_hbm.at[pl.ds(0, 8 * CH)], gv.at[sl],
                              isem).wait()
        pltpu.make_async_copy(b8_hbm.at[pl.ds(0, 8 * CH)], bv.at[sl],
                              isem).wait()
        pltpu.make_async_copy(dst_hbm.at[pl.ds(0, CH)], dv.at[sl],
                              isem).wait()

    def issue_gather(sl):
        pltpu.async_copy(xk_hbm.at[gv.at[sl]], rows.at[sl], rsem)

    def wait_gather(sl):
        pltpu.make_async_copy(xk_hbm.at[pl.ds(0, 8 * CH)], rows.at[sl],
                              rsem).wait()

    def drain_scatter(sl):
        pltpu.make_async_copy(msg.at[sl], acc.at[pl.ds(0, CH)], ssem).wait()

    def compute(sl):
        b16s = [bv[sl, pl.ds(k * CH, 16)] for k in range(8)]
        dsc[sl, :] = dv[sl, :]

        def edge(j, _):
            jdx = jnp.zeros((16,), jnp.int32) + j
            regs = [zero16] * (D // 16)
            for k in range(8):
                bb = lax.gather(
                    b16s[k], jdx[:, None],
                    lax.GatherDimensionNumbers(
                        offset_dims=(), collapsed_slice_dims=(0,),
                        start_index_map=(0,)),
                    (1,),
                    mode=lax.GatherScatterMode.PROMISE_IN_BOUNDS)
                for j2 in range(D // 16):
                    regs[j2] = regs[j2] + bb * rows[sl, k * CH + j,
                                                    pl.ds(j2 * 16, 16)]
            for j2 in range(D // 16):
                msg[sl, j, pl.ds(j2 * 16, 16)] = regs[j2]
            return 0

        lax.fori_loop(0, CH, edge, 0)

    def issue_scatter(sl):
        pltpu.async_copy(msg.at[sl], acc.at[dsc.at[sl]], ssem, add=True)

    fetch_idx(0, 0)
    wait_idx(0)
    issue_gather(0)
    fetch_idx(1, 1)

    def pair(p, _):
        for par in (0, 1):
            other = 1 - par
            ci = 2 * p + par
            wait_gather(par)
            wait_idx(other)
            issue_gather(other)

            @pl.when(p > 0)
            def _():
                drain_scatter(par)

            compute(par)
            issue_scatter(par)
            fetch_idx(jnp.minimum(ci + 2, NCHUNK - 1), par)
        return 0

    lax.fori_loop(0, NCHUNK // 2, pair, 0)

    wait_gather(0)
    wait_idx(1)
    drain_scatter(0)
    compute(0)
    issue_scatter(0)
    drain_scatter(1)
    drain_scatter(0)
    plsc.subcore_barrier()

    def ocopy(k, _):
        r0 = s * RPT + k * ZR
        pltpu.sync_copy(acc.at[pl.ds(r0, ZR)], bounce)
        pltpu.sync_copy(bounce, out_hbm.at[c].at[pl.ds(r0, ZR)])
        return 0

    lax.fori_loop(0, RPT // ZR, ocopy, 0)


def _sc_mp(xk_flat, g8, b8, dst):
    mesh = plsc.VectorSubcoreMesh(core_axis_name="c", subcore_axis_name="s")
    f = pl.kernel(
        _sc_body,
        out_type=jax.ShapeDtypeStruct((NCORE, NP, D), jnp.float32),
        mesh=mesh,
        scratch_types=[
            pltpu.VMEM((2, 8 * CH), jnp.int32),
            pltpu.VMEM((2, 8 * CH), jnp.float32),
            pltpu.VMEM((2, CH), jnp.int32),
            pltpu.VMEM((2, CH), jnp.int32),
            pltpu.VMEM((2, 8 * CH, D), jnp.float32),
            pltpu.VMEM((2, CH, D), jnp.float32),
            pltpu.VMEM((ZR, D), jnp.float32),
            pltpu.VMEM_SHARED((NP, D), jnp.float32),
            pltpu.SemaphoreType.DMA,
            pltpu.SemaphoreType.DMA,
            pltpu.SemaphoreType.DMA,
        ],
    )
    return f(xk_flat, g8, b8, dst)


def _deg_body(dst_hbm, out_hbm, dv, ones_b, bounce, acc, sem):
    c = lax.axis_index("c")
    s = lax.axis_index("s")
    wid = s * NCORE + c
    zero16 = jnp.zeros((16,), jnp.float32)
    one16 = jnp.ones((16,), jnp.float32)

    def zrow(r, _):
        for j in range(D // 16):
            bounce[r, pl.ds(j * 16, 16)] = zero16
        return 0

    lax.fori_loop(0, ZR, zrow, 0)

    def orow(r, _):
        for j in range(D // 16):
            ones_b[r, pl.ds(j * 16, 16)] = one16
        return 0

    lax.fori_loop(0, DCH, orow, 0)

    def zcopy(k, _):
        pltpu.sync_copy(bounce, acc.at[pl.ds(s * RPT + k * ZR, ZR)])
        return 0

    lax.fori_loop(0, RPT // ZR, zcopy, 0)
    plsc.subcore_barrier()

    def chunk(i, _):
        base = wid * EPT + i * DCH
        pltpu.sync_copy(dst_hbm.at[pl.ds(base, DCH)], dv)
        pltpu.sync_copy(ones_b, acc.at[dv], add=True)
        return 0

    lax.fori_loop(0, DNCHUNK, chunk, 0)
    plsc.subcore_barrier()

    def ocopy(k, _):
        r0 = s * RPT + k * ZR
        pltpu.sync_copy(acc.at[pl.ds(r0, ZR)], bounce)
        pltpu.sync_copy(bounce, out_hbm.at[c].at[pl.ds(r0, ZR)])
        return 0

    lax.fori_loop(0, RPT // ZR, ocopy, 0)


def _sc_deg(dst):
    mesh = plsc.VectorSubcoreMesh(core_axis_name="c", subcore_axis_name="s")
    f = pl.kernel(
        _deg_body,
        out_type=jax.ShapeDtypeStruct((NCORE, NP, D), jnp.float32),
        mesh=mesh,
        scratch_types=[
            pltpu.VMEM((DCH,), jnp.int32),
            pltpu.VMEM((DCH, D), jnp.float32),
            pltpu.VMEM((ZR, D), jnp.float32),
            pltpu.VMEM_SHARED((NP, D), jnp.float32),
            pltpu.SemaphoreType.DMA,
        ],
    )
    return f(dst)


def _combine_body(acc_ref, dacc_ref, x_ref, root_ref, bias_ref, g_ref,
                  be_ref, out_ref):
    a = acc_ref[0][:N, :] + acc_ref[1][:N, :]
    deg = jnp.maximum(dacc_ref[0][:N, :1] + dacc_ref[1][:N, :1], 1.0)
    agg = a / deg
    z = agg + jnp.dot(x_ref[...], root_ref[...],
                      preferred_element_type=jnp.float32) + bias_ref[...]
    z = jnp.maximum(z, 0.0)
    m = jnp.mean(z, axis=0, keepdims=True)
    v = jnp.mean((z - m) ** 2, axis=0, keepdims=True)
    out_ref[...] = (z - m) * lax.rsqrt(v + 1e-5) * g_ref[...] + be_ref[...]


def _combine(acc, dacc, x, root, bias, g, be):
    return pl.pallas_call(
        _combine_body,
        out_shape=jax.ShapeDtypeStruct((N, D), jnp.float32),
    )(acc, dacc, x, root, bias[None, :], g[None, :], be[None, :])


def _head_body(h_ref, batch_ref, L1w_ref, L1b_ref, g3_ref, be3_ref,
               F1w_ref, F1b_ref, g4_ref, be4_ref, F2w_ref, F2b_ref,
               g5_ref, be5_ref, F3w_ref, F3b_ref, out_ref):
    h = h_ref[...]
    z = jnp.maximum(jnp.dot(h, L1w_ref[...],
                            preferred_element_type=jnp.float32)
                    + L1b_ref[...], 0.0)
    m = jnp.mean(z, axis=0, keepdims=True)
    v = jnp.mean((z - m) ** 2, axis=0, keepdims=True)
    z = (z - m) * lax.rsqrt(v + 1e-5) * g3_ref[...] + be3_ref[...]
    batch = batch_ref[...]
    neg = jnp.float32(-3.0e38)
    rows = []
    for g in range(NUM_GRAPHS):
        mask = (batch == g)
        rows.append(jnp.max(jnp.where(mask, z, neg), axis=0)[None, :])
    pooled = jnp.concatenate(rows, axis=0)

    def bn_small(o, gr, ber):
        mm = jnp.mean(o, axis=0, keepdims=True)
        vv = jnp.mean((o - mm) ** 2, axis=0, keepdims=True)
        return (o - mm) * lax.rsqrt(vv + 1e-5) * gr[...] + ber[...]

    o = jnp.maximum(jnp.dot(pooled, F1w_ref[...],
                            preferred_element_type=jnp.float32)
                    + F1b_ref[...], 0.0)
    o = bn_small(o, g4_ref, be4_ref)
    o = jnp.maximum(jnp.dot(o, F2w_ref[...],
                            preferred_element_type=jnp.float32)
                    + F2b_ref[...], 0.0)
    o = bn_small(o, g5_ref, be5_ref)
    o = jnp.dot(o, F3w_ref[...], preferred_element_type=jnp.float32) \
        + F3b_ref[...]
    omax = jnp.max(o, axis=1, keepdims=True)
    lse = jnp.log(jnp.sum(jnp.exp(o - omax), axis=1, keepdims=True)) + omax
    out_ref[...] = o - lse


def _head(h, batch, L1w, L1b, g3, be3, F1w, F1b, g4, be4, F2w, F2b,
          g5, be5, F3w, F3b):
    return pl.pallas_call(
        _head_body,
        out_shape=jax.ShapeDtypeStruct((NUM_GRAPHS, N_CLASSES), jnp.float32),
    )(h, batch[:, None], L1w, L1b[None, :], g3[None, :], be3[None, :],
      F1w, F1b[None, :], g4[None, :], be4[None, :], F2w, F2b[None, :],
      g5[None, :], be5[None, :], F3w, F3b[None, :])


def kernel(x, edge_index, pseudo, batch, W1, root1, b1, g1, be1, W2, root2,
           b2, g2, be2, L1w, L1b, g3, be3, F1w, F1b, g4, be4, F2w, F2b,
           g5, be5, F3w, F3b):
    src = edge_index[0]
    dst = edge_index[1]
    ps8 = jnp.concatenate(
        [pseudo.T, src.astype(jnp.float32)[None, :],
         jnp.zeros((4, E), jnp.float32)], axis=0)
    b8, g8 = _eprep(ps8)
    b8f = b8.reshape(8, NWORK, NCHUNK, CH).transpose(1, 2, 0, 3).reshape(-1)
    g8f = g8.reshape(8, NWORK, NCHUNK, CH).transpose(1, 2, 0, 3).reshape(-1)

    dacc = _sc_deg(dst)

    def layer(xin, W, root, bias, g, be):
        xk = _xk(xin, W).reshape(KD * N, D)
        acc = _sc_mp(xk, g8f, b8f, dst)
        return _combine(acc, dacc, xin, root, bias, g, be)

    x1 = layer(x, W1, root1, b1, g1, be1)
    x2 = layer(x1, W2, root2, b2, g2, be2)
    h = jnp.concatenate([x1, x2], axis=1)
    return _head(h, batch, L1w, L1b, g3, be3, F1w, F1b, g4, be4,
                 F2w, F2b, g5, be5, F3w, F3b)


# speedup vs baseline: 2.1121x; 1.0009x over previous
import functools

import jax
import jax.numpy as jnp
from jax import lax
from jax.experimental import pallas as pl
from jax.experimental.pallas import tpu as pltpu
from jax.experimental.pallas import tpu_sc as plsc

N = 10000
E = 320000
D = 128
KD = 27
NUM_GRAPHS = 8
N_CLASSES = 10
NCORE = 2
NSUB = 16
NWORK = NCORE * NSUB
EPT = E // NWORK
CH = 16
NCHUNK = EPT // CH
DCH = 80
DNCHUNK = EPT // DCH
NP = 10240
RPT = NP // NSUB
ZR = 32


def _xk_body(x_ref, w_ref, out_ref):
    out_ref[0] = jnp.dot(x_ref[...], w_ref[0],
                         preferred_element_type=jnp.float32)


def _xk(x, W):
    return pl.pallas_call(
        _xk_body,
        grid=(KD,),
        in_specs=[
            pl.BlockSpec((N, D), lambda k: (0, 0)),
            pl.BlockSpec((1, D, D), lambda k: (k, 0, 0)),
        ],
        out_specs=pl.BlockSpec((1, N, D), lambda k: (k, 0, 0)),
        out_shape=jax.ShapeDtypeStruct((KD, N, D), jnp.float32),
    )(x, W)


_EB = 6400


def _eprep_body(ps_ref, b_ref, g_ref):
    ps = ps_ref[...]
    p = ps * jnp.float32(2.0)
    lo = jnp.clip(jnp.floor(p), 0.0, 1.0)
    fr = p - lo
    srcf = ps_ref[3:4, :]
    for s in range(8):
        bx, by, bz = (s >> 0) & 1, (s >> 1) & 1, (s >> 2) & 1
        w = jnp.ones_like(srcf)
        wi = jnp.zeros_like(srcf)
        for d, bit in enumerate((bx, by, bz)):
            frd = fr[d:d + 1, :]
            lod = lo[d:d + 1, :]
            w = w * (frd if bit else (1.0 - frd))
            wi = wi + (lod + jnp.float32(bit)) * jnp.float32(3 ** d)
        gidx = wi * jnp.float32(N) + srcf
        b_ref[s:s + 1, :] = w
        g_ref[s:s + 1, :] = gidx.astype(jnp.int32)


def _eprep(ps8):
    return pl.pallas_call(
        _eprep_body,
        grid=(E // _EB,),
        in_specs=[pl.BlockSpec((8, _EB), lambda j: (0, j))],
        out_specs=[pl.BlockSpec((8, _EB), lambda j: (0, j)),
                   pl.BlockSpec((8, _EB), lambda j: (0, j))],
        out_shape=[jax.ShapeDtypeStruct((8, E), jnp.float32),
                   jax.ShapeDtypeStruct((8, E), jnp.int32)],
    )(ps8)


def _sc_body(xk_hbm, g8_hbm, b8_hbm, dst_hbm, out_hbm,
             gv, bv, dv, dsc, rows, msg, bounce, acc, isem, rsem, ssem):
    c = lax.axis_index("c")
    s = lax.axis_index("s")
    wid = s * NCORE + c
    zero16 = jnp.zeros((16,), jnp.float32)

    def zrow(r, _):
        for j in range(D // 16):
            bounce[r, pl.ds(j * 16, 16)] = zero16
        return 0

    lax.fori_loop(0, ZR, zrow, 0)

    def zcopy(k, _):
        pltpu.sync_copy(bounce, acc.at[pl.ds(s * RPT + k * ZR, ZR)])
        return 0

    lax.fori_loop(0, RPT // ZR, zcopy, 0)
    plsc.subcore_barrier()

    def fetch_idx(ci, sl):
        base = wid * EPT + ci * CH
        fbase = (wid * NCHUNK + ci) * 8 * CH
        pltpu.async_copy(g8_hbm.at[pl.ds(fbase, 8 * CH)], gv.at[sl], isem)
        pltpu.async_copy(b8_hbm.at[pl.ds(fbase, 8 * CH)], bv.at[sl], isem)
        pltpu.async_copy(dst_hbm.at[pl.ds(base, CH)], dv.at[sl], isem)

    def wait_idx(sl):
        pltpu.make_async_copy(g8_hbm.at[pl.ds(0, 8 * CH)], gv.at[sl],
                              isem).wait()
        pltpu.make_async_copy(b8_hbm.at[pl.ds(0, 8 * CH)], bv.at[sl],
                              isem).wait()
        pltpu.make_async_copy(dst_hbm.at[pl.ds(0, CH)], dv.at[sl],
                              isem).wait()

    def issue_gather(sl):
        pltpu.async_copy(xk_hbm.at[gv.at[sl]], rows.at[sl], rsem)

    def wait_gather(sl):
        pltpu.make_async_copy(xk_hbm.at[pl.ds(0, 8 * CH)], rows.at[sl],
                              rsem).wait()

    def drain_scatter(sl):
        pltpu.make_async_copy(msg.at[sl], acc.at[pl.ds(0, CH)], ssem).wait()

    def compute(sl):
        b16s = [bv[sl, pl.ds(k * CH, 16)] for k in range(8)]
        dsc[sl, :] = dv[sl, :]

        def edge4(g, _):
            for u in range(4):
                j = g * 4 + u
                jdx = jnp.zeros((16,), jnp.int32) + j
                regs = [zero16] * (D // 16)
                for k in range(8):
                    bb = lax.gather(
                        b16s[k], jdx[:, None],
                        lax.GatherDimensionNumbers(
                            offset_dims=(), collapsed_slice_dims=(0,),
                            start_index_map=(0,)),
                        (1,),
                        mode=lax.GatherScatterMode.PROMISE_IN_BOUNDS)
                    for j2 in range(D // 16):
                        regs[j2] = regs[j2] + bb * rows[sl, k * CH + j,
                                                        pl.ds(j2 * 16, 16)]
                for j2 in range(D // 16):
                    msg[sl, j, pl.ds(j2 * 16, 16)] = regs[j2]
            return 0

        lax.fori_loop(0, CH // 4, edge4, 0)

    def issue_scatter(sl):
        pltpu.async_copy(msg.at[sl], acc.at[dsc.at[sl]], ssem, add=True)

    fetch_idx(0, 0)
    wait_idx(0)
    issue_gather(0)
    fetch_idx(1, 1)

    def pair(p, _):
        for par in (0, 1):
            other = 1 - par
            ci = 2 * p + par
            wait_gather(par)
            wait_idx(other)
            issue_gather(other)

            @pl.when(p > 0)
            def _():
                drain_scatter(par)

            compute(par)
            issue_scatter(par)
            fetch_idx(jnp.minimum(ci + 2, NCHUNK - 1), par)
        return 0

    lax.fori_loop(0, NCHUNK // 2, pair, 0)

    wait_gather(0)
    wait_idx(1)
    drain_scatter(0)
    compute(0)
    issue_scatter(0)
    drain_scatter(1)
    drain_scatter(0)
    plsc.subcore_barrier()

    def ocopy(k, _):
        r0 = s * RPT + k * ZR
        pltpu.sync_copy(acc.at[pl.ds(r0, ZR)], bounce)
        pltpu.sync_copy(bounce, out_hbm.at[c].at[pl.ds(r0, ZR)])
        return 0

    lax.fori_loop(0, RPT // ZR, ocopy, 0)


def _sc_mp(xk_flat, g8, b8, dst):
    mesh = plsc.VectorSubcoreMesh(core_axis_name="c", subcore_axis_name="s")
    f = pl.kernel(
        _sc_body,
        out_type=jax.ShapeDtypeStruct((NCORE, NP, D), jnp.float32),
        mesh=mesh,
        scratch_types=[
            pltpu.VMEM((2, 8 * CH), jnp.int32),
            pltpu.VMEM((2, 8 * CH), jnp.float32),
            pltpu.VMEM((2, CH), jnp.int32),
            pltpu.VMEM((2, CH), jnp.int32),
            pltpu.VMEM((2, 8 * CH, D), jnp.float32),
            pltpu.VMEM((2, CH, D), jnp.float32),
            pltpu.VMEM((ZR, D), jnp.float32),
            pltpu.VMEM_SHARED((NP, D), jnp.float32),
            pltpu.SemaphoreType.DMA,
            pltpu.SemaphoreType.DMA,
            pltpu.SemaphoreType.DMA,
        ],
    )
    return f(xk_flat, g8, b8, dst)


def _deg_body(dst_hbm, out_hbm, dv, ones_b, bounce, acc, sem):
    c = lax.axis_index("c")
    s = lax.axis_index("s")
    wid = s * NCORE + c
    zero16 = jnp.zeros((16,), jnp.float32)
    one16 = jnp.ones((16,), jnp.float32)

    def zrow(r, _):
        for j in range(D // 16):
            bounce[r, pl.ds(j * 16, 16)] = zero16
        return 0

    lax.fori_loop(0, ZR, zrow, 0)

    def orow(r, _):
        for j in range(D // 16):
            ones_b[r, pl.ds(j * 16, 16)] = one16
        return 0

    lax.fori_loop(0, DCH, orow, 0)

    def zcopy(k, _):
        pltpu.sync_copy(bounce, acc.at[pl.ds(s * RPT + k * ZR, ZR)])
        return 0

    lax.fori_loop(0, RPT // ZR, zcopy, 0)
    plsc.subcore_barrier()

    def chunk(i, _):
        base = wid * EPT + i * DCH
        pltpu.sync_copy(dst_hbm.at[pl.ds(base, DCH)], dv)
        pltpu.sync_copy(ones_b, acc.at[dv], add=True)
        return 0

    lax.fori_loop(0, DNCHUNK, chunk, 0)
    plsc.subcore_barrier()

    def ocopy(k, _):
        r0 = s * RPT + k * ZR
        pltpu.sync_copy(acc.at[pl.ds(r0, ZR)], bounce)
        pltpu.sync_copy(bounce, out_hbm.at[c].at[pl.ds(r0, ZR)])
        return 0

    lax.fori_loop(0, RPT // ZR, ocopy, 0)


def _sc_deg(dst):
    mesh = plsc.VectorSubcoreMesh(core_axis_name="c", subcore_axis_name="s")
    f = pl.kernel(
        _deg_body,
        out_type=jax.ShapeDtypeStruct((NCORE, NP, D), jnp.float32),
        mesh=mesh,
        scratch_types=[
            pltpu.VMEM((DCH,), jnp.int32),
            pltpu.VMEM((DCH, D), jnp.float32),
            pltpu.VMEM((ZR, D), jnp.float32),
            pltpu.VMEM_SHARED((NP, D), jnp.float32),
            pltpu.SemaphoreType.DMA,
        ],
    )
    return f(dst)


def _combine_body(acc_ref, dacc_ref, x_ref, root_ref, bias_ref, g_ref,
                  be_ref, out_ref):
    a = acc_ref[0][:N, :] + acc_ref[1][:N, :]
    deg = jnp.maximum(dacc_ref[0][:N, :1] + dacc_ref[1][:N, :1], 1.0)
    agg = a / deg
    z = agg + jnp.dot(x_ref[...], root_ref[...],
                      preferred_element_type=jnp.float32) + bias_ref[...]
    z = jnp.maximum(z, 0.0)
    m = jnp.mean(z, axis=0, keepdims=True)
    v = jnp.mean((z - m) ** 2, axis=0, keepdims=True)
    out_ref[...] = (z - m) * lax.rsqrt(v + 1e-5) * g_ref[...] + be_ref[...]


def _combine(acc, dacc, x, root, bias, g, be):
    return pl.pallas_call(
        _combine_body,
        out_shape=jax.ShapeDtypeStruct((N, D), jnp.float32),
    )(acc, dacc, x, root, bias[None, :], g[None, :], be[None, :])


def _head_body(h_ref, batch_ref, L1w_ref, L1b_ref, g3_ref, be3_ref,
               F1w_ref, F1b_ref, g4_ref, be4_ref, F2w_ref, F2b_ref,
               g5_ref, be5_ref, F3w_ref, F3b_ref, out_ref):
    h = h_ref[...]
    z = jnp.maximum(jnp.dot(h, L1w_ref[...],
                            preferred_element_type=jnp.float32)
                    + L1b_ref[...], 0.0)
    m = jnp.mean(z, axis=0, keepdims=True)
    v = jnp.mean((z - m) ** 2, axis=0, keepdims=True)
    z = (z - m) * lax.rsqrt(v + 1e-5) * g3_ref[...] + be3_ref[...]
    batch = batch_ref[...]
    neg = jnp.float32(-3.0e38)
    rows = []
    for g in range(NUM_GRAPHS):
        mask = (batch == g)
        rows.append(jnp.max(jnp.where(mask, z, neg), axis=0)[None, :])
    pooled = jnp.concatenate(rows, axis=0)

    def bn_small(o, gr, ber):
        mm = jnp.mean(o, axis=0, keepdims=True)
        vv = jnp.mean((o - mm) ** 2, axis=0, keepdims=True)
        return (o - mm) * lax.rsqrt(vv + 1e-5) * gr[...] + ber[...]

    o = jnp.maximum(jnp.dot(pooled, F1w_ref[...],
                            preferred_element_type=jnp.float32)
                    + F1b_ref[...], 0.0)
    o = bn_small(o, g4_ref, be4_ref)
    o = jnp.maximum(jnp.dot(o, F2w_ref[...],
                            preferred_element_type=jnp.float32)
                    + F2b_ref[...], 0.0)
    o = bn_small(o, g5_ref, be5_ref)
    o = jnp.dot(o, F3w_ref[...], preferred_element_type=jnp.float32) \
        + F3b_ref[...]
    omax = jnp.max(o, axis=1, keepdims=True)
    lse = jnp.log(jnp.sum(jnp.exp(o - omax), axis=1, keepdims=True)) + omax
    out_ref[...] = o - lse


def _head(h, batch, L1w, L1b, g3, be3, F1w, F1b, g4, be4, F2w, F2b,
          g5, be5, F3w, F3b):
    return pl.pallas_call(
        _head_body,
        out_shape=jax.ShapeDtypeStruct((NUM_GRAPHS, N_CLASSES), jnp.float32),
    )(h, batch[:, None], L1w, L1b[None, :], g3[None, :], be3[None, :],
      F1w, F1b[None, :], g4[None, :], be4[None, :], F2w, F2b[None, :],
      g5[None, :], be5[None, :], F3w, F3b[None, :])


def kernel(x, edge_index, pseudo, batch, W1, root1, b1, g1, be1, W2, root2,
           b2, g2, be2, L1w, L1b, g3, be3, F1w, F1b, g4, be4, F2w, F2b,
           g5, be5, F3w, F3b):
    src = edge_index[0]
    dst = edge_index[1]
    ps8 = jnp.concatenate(
        [pseudo.T, src.astype(jnp.float32)[None, :],
         jnp.zeros((4, E), jnp.float32)], axis=0)
    b8, g8 = _eprep(ps8)
    b8f = b8.reshape(8, NWORK, NCHUNK, CH).transpose(1, 2, 0, 3).reshape(-1)
    g8f = g8.reshape(8, NWORK, NCHUNK, CH).transpose(1, 2, 0, 3).reshape(-1)

    dacc = _sc_deg(dst)

    def layer(xin, W, root, bias, g, be):
        xk = _xk(xin, W).reshape(KD * N, D)
        acc = _sc_mp(xk, g8f, b8f, dst)
        return _combine(acc, dacc, xin, root, bias, g, be)

    x1 = layer(x, W1, root1, b1, g1, be1)
    x2 = layer(x1, W2, root2, b2, g2, be2)
    h = jnp.concatenate([x1, x2], axis=1)
    return _head(h, batch, L1w, L1b, g3, be3, F1w, F1b, g4, be4,
                 F2w, F2b, g5, be5, F3w, F3b)


# speedup vs baseline: 2.1155x; 1.0016x over previous
import functools

import jax
import jax.numpy as jnp
import numpy as np
from jax import lax
from jax.experimental import pallas as pl
from jax.experimental.pallas import tpu as pltpu
from jax.experimental.pallas import tpu_sc as plsc

N = 10000
E = 320000
D = 128
KD = 27
NUM_GRAPHS = 8
N_CLASSES = 10
NCORE = 2
NSUB = 16
NWORK = NCORE * NSUB
EPT = E // NWORK
CH = 16
NCHUNK = EPT // CH
DCH = 80
DNCHUNK = EPT // DCH
NP = 10240
RPT = NP // NSUB
ZR = 32


def _xk_body(x_ref, w_ref, out_ref):
    out_ref[0] = jnp.dot(x_ref[...], w_ref[0],
                         preferred_element_type=jnp.float32)


def _xk(x, W):
    return pl.pallas_call(
        _xk_body,
        grid=(KD,),
        in_specs=[
            pl.BlockSpec((N, D), lambda k: (0, 0)),
            pl.BlockSpec((1, D, D), lambda k: (k, 0, 0)),
        ],
        out_specs=pl.BlockSpec((1, N, D), lambda k: (k, 0, 0)),
        out_shape=jax.ShapeDtypeStruct((KD, N, D), jnp.float32),
    )(x, W)


_EB = 6400


def _eprep_body(ps_ref, b_ref, g_ref):
    ps = ps_ref[...]
    p = ps * jnp.float32(2.0)
    lo = jnp.clip(jnp.floor(p), 0.0, 1.0)
    fr = p - lo
    srcf = ps_ref[3:4, :]
    for s in range(8):
        bx, by, bz = (s >> 0) & 1, (s >> 1) & 1, (s >> 2) & 1
        w = jnp.ones_like(srcf)
        wi = jnp.zeros_like(srcf)
        for d, bit in enumerate((bx, by, bz)):
            frd = fr[d:d + 1, :]
            lod = lo[d:d + 1, :]
            w = w * (frd if bit else (1.0 - frd))
            wi = wi + (lod + jnp.float32(bit)) * jnp.float32(3 ** d)
        gidx = wi * jnp.float32(N) + srcf
        b_ref[s:s + 1, :] = w
        g_ref[s:s + 1, :] = gidx.astype(jnp.int32)


def _eprep(ps8):
    return pl.pallas_call(
        _eprep_body,
        grid=(E // _EB,),
        in_specs=[pl.BlockSpec((8, _EB), lambda j: (0, j))],
        out_specs=[pl.BlockSpec((8, _EB), lambda j: (0, j)),
                   pl.BlockSpec((8, _EB), lambda j: (0, j))],
        out_shape=[jax.ShapeDtypeStruct((8, E), jnp.float32),
                   jax.ShapeDtypeStruct((8, E), jnp.int32)],
    )(ps8)


def _sc_body(xk_hbm, g8_hbm, b8_hbm, dst_hbm, out_hbm,
             gv, bv, dv, dsc, rows, msg, bounce, acc, isem, rsem, ssem):
    c = lax.axis_index("c")
    s = lax.axis_index("s")
    wid = s * NCORE + c
    zero16 = jnp.zeros((16,), jnp.float32)

    def zrow(r, _):
        for j in range(D // 16):
            bounce[r, pl.ds(j * 16, 16)] = zero16
        return 0

    lax.fori_loop(0, ZR, zrow, 0)

    def zcopy(k, _):
        pltpu.sync_copy(bounce, acc.at[pl.ds(s * RPT + k * ZR, ZR)])
        return 0

    lax.fori_loop(0, RPT // ZR, zcopy, 0)
    plsc.subcore_barrier()

    def fetch_idx(ci, sl):
        base = wid * EPT + ci * CH
        fbase = (wid * NCHUNK + ci) * 8 * CH
        pltpu.async_copy(g8_hbm.at[pl.ds(fbase, 8 * CH)], gv.at[sl], isem)
        pltpu.async_copy(b8_hbm.at[pl.ds(fbase, 8 * CH)], bv.at[sl], isem)
        pltpu.async_copy(dst_hbm.at[pl.ds(base, CH)], dv.at[sl], isem)

    def wait_idx(sl):
        pltpu.make_async_copy(g8_hbm.at[pl.ds(0, 8 * CH)], gv.at[sl],
                              isem).wait()
        pltpu.make_async_copy(b8_hbm.at[pl.ds(0, 8 * CH)], bv.at[sl],
                              isem).wait()
        pltpu.make_async_copy(dst_hbm.at[pl.ds(0, CH)], dv.at[sl],
                              isem).wait()

    def issue_gather(sl):
        pltpu.async_copy(xk_hbm.at[gv.at[sl]], rows.at[sl], rsem)

    def wait_gather(sl):
        pltpu.make_async_copy(xk_hbm.at[pl.ds(0, 8 * CH)], rows.at[sl],
                              rsem).wait()

    def drain_scatter(sl):
        pltpu.make_async_copy(msg.at[sl], acc.at[pl.ds(0, CH)], ssem).wait()

    def compute(sl):
        b16s = [bv[sl, pl.ds(k * CH, 16)] for k in range(8)]
        dsc[sl, :] = dv[sl, :]

        def edge4(g, _):
            for u in range(4):
                j = g * 4 + u
                jdx = jnp.zeros((16,), jnp.int32) + j
                regs = [zero16] * (D // 16)
                for k in range(8):
                    bb = lax.gather(
                        b16s[k], jdx[:, None],
                        lax.GatherDimensionNumbers(
                            offset_dims=(), collapsed_slice_dims=(0,),
                            start_index_map=(0,)),
                        (1,),
                        mode=lax.GatherScatterMode.PROMISE_IN_BOUNDS)
                    for j2 in range(D // 16):
                        regs[j2] = regs[j2] + bb * rows[sl, k * CH + j,
                                                        pl.ds(j2 * 16, 16)]
                for j2 in range(D // 16):
                    msg[sl, j, pl.ds(j2 * 16, 16)] = regs[j2]
            return 0

        lax.fori_loop(0, CH // 4, edge4, 0)

    def issue_scatter(sl):
        pltpu.async_copy(msg.at[sl], acc.at[dsc.at[sl]], ssem, add=True)

    fetch_idx(0, 0)
    wait_idx(0)
    issue_gather(0)
    fetch_idx(1, 1)

    def pair(p, _):
        for par in (0, 1):
            other = 1 - par
            ci = 2 * p + par
            wait_gather(par)
            wait_idx(other)
            issue_gather(other)

            @pl.when(p > 0)
            def _():
                drain_scatter(par)

            compute(par)
            issue_scatter(par)
            fetch_idx(jnp.minimum(ci + 2, NCHUNK - 1), par)
        return 0

    lax.fori_loop(0, NCHUNK // 2, pair, 0)

    wait_gather(0)
    wait_idx(1)
    drain_scatter(0)
    compute(0)
    issue_scatter(0)
    drain_scatter(1)
    drain_scatter(0)
    plsc.subcore_barrier()

    def ocopy(k, _):
        r0 = s * RPT + k * ZR
        pltpu.sync_copy(acc.at[pl.ds(r0, ZR)], bounce)
        pltpu.sync_copy(bounce, out_hbm.at[c].at[pl.ds(r0, ZR)])
        return 0

    lax.fori_loop(0, RPT // ZR, ocopy, 0)


def _sc_mp(xk_flat, g8, b8, dst):
    mesh = plsc.VectorSubcoreMesh(core_axis_name="c", subcore_axis_name="s")
    f = pl.kernel(
        _sc_body,
        out_type=jax.ShapeDtypeStruct((NCORE, NP, D), jnp.float32),
        mesh=mesh,
        scratch_types=[
            pltpu.VMEM((2, 8 * CH), jnp.int32),
            pltpu.VMEM((2, 8 * CH), jnp.float32),
            pltpu.VMEM((2, CH), jnp.int32),
            pltpu.VMEM((2, CH), jnp.int32),
            pltpu.VMEM((2, 8 * CH, D), jnp.float32),
            pltpu.VMEM((2, CH, D), jnp.float32),
            pltpu.VMEM((ZR, D), jnp.float32),
            pltpu.VMEM_SHARED((NP, D), jnp.float32),
            pltpu.SemaphoreType.DMA,
            pltpu.SemaphoreType.DMA,
            pltpu.SemaphoreType.DMA,
        ],
    )
    return f(xk_flat, g8, b8, dst)


def _deg_body(dst_hbm, out_hbm, dv, ones_b, bounce, acc, sem):
    c = lax.axis_index("c")
    s = lax.axis_index("s")
    wid = s * NCORE + c
    zero16 = jnp.zeros((16,), jnp.float32)
    one16 = jnp.ones((16,), jnp.float32)

    def zrow(r, _):
        for j in range(D // 16):
            bounce[r, pl.ds(j * 16, 16)] = zero16
        return 0

    lax.fori_loop(0, ZR, zrow, 0)

    def orow(r, _):
        for j in range(D // 16):
            ones_b[r, pl.ds(j * 16, 16)] = one16
        return 0

    lax.fori_loop(0, DCH, orow, 0)

    def zcopy(k, _):
        pltpu.sync_copy(bounce, acc.at[pl.ds(s * RPT + k * ZR, ZR)])
        return 0

    lax.fori_loop(0, RPT // ZR, zcopy, 0)
    plsc.subcore_barrier()

    def chunk(i, _):
        base = wid * EPT + i * DCH
        pltpu.sync_copy(dst_hbm.at[pl.ds(base, DCH)], dv)
        pltpu.sync_copy(ones_b, acc.at[dv], add=True)
        return 0

    lax.fori_loop(0, DNCHUNK, chunk, 0)
    plsc.subcore_barrier()

    def ocopy(k, _):
        r0 = s * RPT + k * ZR
        pltpu.sync_copy(acc.at[pl.ds(r0, ZR)], bounce)
        pltpu.sync_copy(bounce, out_hbm.at[c].at[pl.ds(r0, ZR)])
        return 0

    lax.fori_loop(0, RPT // ZR, ocopy, 0)


def _sc_deg(dst):
    mesh = plsc.VectorSubcoreMesh(core_axis_name="c", subcore_axis_name="s")
    f = pl.kernel(
        _deg_body,
        out_type=jax.ShapeDtypeStruct((NCORE, NP, D), jnp.float32),
        mesh=mesh,
        scratch_types=[
            pltpu.VMEM((DCH,), jnp.int32),
            pltpu.VMEM((DCH, D), jnp.float32),
            pltpu.VMEM((ZR, D), jnp.float32),
            pltpu.VMEM_SHARED((NP, D), jnp.float32),
            pltpu.SemaphoreType.DMA,
        ],
    )
    return f(dst)


def _combine_body(acc_ref, dacc_ref, x_ref, root_ref, bias_ref, g_ref,
                  be_ref, out_ref):
    a = acc_ref[0][:N, :] + acc_ref[1][:N, :]
    deg = jnp.maximum(dacc_ref[0][:N, :1] + dacc_ref[1][:N, :1], 1.0)
    agg = a / deg
    z = agg + jnp.dot(x_ref[...], root_ref[...],
                      preferred_element_type=jnp.float32) + bias_ref[...]
    z = jnp.maximum(z, 0.0)
    m = jnp.mean(z, axis=0, keepdims=True)
    v = jnp.mean((z - m) ** 2, axis=0, keepdims=True)
    out_ref[...] = (z - m) * lax.rsqrt(v + 1e-5) * g_ref[...] + be_ref[...]


def _combine(acc, dacc, x, root, bias, g, be):
    return pl.pallas_call(
        _combine_body,
        out_shape=jax.ShapeDtypeStruct((N, D), jnp.float32),
    )(acc, dacc, x, root, bias[None, :], g[None, :], be[None, :])


def _head_body(h_ref, batch_ref, L1w_ref, L1b_ref, g3_ref, be3_ref,
               F1w_ref, F1b_ref, g4_ref, be4_ref, F2w_ref, F2b_ref,
               g5_ref, be5_ref, F3w_ref, F3b_ref, out_ref):
    h = h_ref[...]
    z = jnp.maximum(jnp.dot(h, L1w_ref[...],
                            preferred_element_type=jnp.float32)
                    + L1b_ref[...], 0.0)
    m = jnp.mean(z, axis=0, keepdims=True)
    v = jnp.mean((z - m) ** 2, axis=0, keepdims=True)
    z = (z - m) * lax.rsqrt(v + 1e-5) * g3_ref[...] + be3_ref[...]
    batch = batch_ref[...]
    neg = jnp.float32(-3.0e38)
    rows = []
    for g in range(NUM_GRAPHS):
        mask = (batch == g)
        rows.append(jnp.max(jnp.where(mask, z, neg), axis=0)[None, :])
    pooled = jnp.concatenate(rows, axis=0)

    def bn_small(o, gr, ber):
        mm = jnp.mean(o, axis=0, keepdims=True)
        vv = jnp.mean((o - mm) ** 2, axis=0, keepdims=True)
        return (o - mm) * lax.rsqrt(vv + 1e-5) * gr[...] + ber[...]

    o = jnp.maximum(jnp.dot(pooled, F1w_ref[...],
                            preferred_element_type=jnp.float32)
                    + F1b_ref[...], 0.0)
    o = bn_small(o, g4_ref, be4_ref)
    o = jnp.maximum(jnp.dot(o, F2w_ref[...],
                            preferred_element_type=jnp.float32)
                    + F2b_ref[...], 0.0)
    o = bn_small(o, g5_ref, be5_ref)
    o = jnp.dot(o, F3w_ref[...], preferred_element_type=jnp.float32) \
        + F3b_ref[...]
    omax = jnp.max(o, axis=1, keepdims=True)
    lse = jnp.log(jnp.sum(jnp.exp(o - omax), axis=1, keepdims=True)) + omax
    out_ref[...] = o - lse


def _head(h, batch, L1w, L1b, g3, be3, F1w, F1b, g4, be4, F2w, F2b,
          g5, be5, F3w, F3b):
    return pl.pallas_call(
        _head_body,
        out_shape=jax.ShapeDtypeStruct((NUM_GRAPHS, N_CLASSES), jnp.float32),
    )(h, batch[:, None], L1w, L1b[None, :], g3[None, :], be3[None, :],
      F1w, F1b[None, :], g4[None, :], be4[None, :], F2w, F2b[None, :],
      g5[None, :], be5[None, :], F3w, F3b[None, :])


def kernel(x, edge_index, pseudo, batch, W1, root1, b1, g1, be1, W2, root2,
           b2, g2, be2, L1w, L1b, g3, be3, F1w, F1b, g4, be4, F2w, F2b,
           g5, be5, F3w, F3b):
    src = edge_index[0]
    dst = edge_index[1]
    ps8 = jnp.concatenate(
        [pseudo.T, src.astype(jnp.float32)[None, :],
         jnp.zeros((4, E), jnp.float32)], axis=0)
    b8, g8 = _eprep(ps8)
    b8f = b8.reshape(8, NWORK, NCHUNK, CH).transpose(1, 2, 0, 3).reshape(-1)
    g8f = g8.reshape(8, NWORK, NCHUNK, CH).transpose(1, 2, 0, 3).reshape(-1)

    dacc = _sc_deg(dst)

    def layer(xin, W, root, bias, g, be):
        xk = _xk(xin, W).reshape(KD * N, D)
        acc = _sc_mp(xk, g8f, b8f, dst)
        return _combine(acc, dacc, xin, root, bias, g, be)

    x1 = layer(x, W1, root1, b1, g1, be1)
    x2 = layer(x1, W2, root2, b2, g2, be2)
    h = jnp.concatenate([x1, x2], axis=1)
    return _head(h, batch, L1w, L1b, g3, be3, F1w, F1b, g4, be4,
                 F2w, F2b, g5, be5, F3w, F3b)
